# baseline (speedup 1.0000x reference)
import functools

import jax
import jax.numpy as jnp
import numpy as np
from jax import lax
from jax.experimental import pallas as pl
from jax.experimental.pallas import tpu as pltpu
from jax.experimental.pallas import tpu_sc as plsc

B = 16384
D = 128
NEG = 20
G = NEG + 1
T = G * B
NW = 32
RPW = B // NW
CH = 128
STEPS = G * (RPW // CH)
DCH = D // 16


def _sc_body(x_hbm, w_hbm, idx_hbm, out_hbm,
             x_v, idx_v, wbuf0, wbuf1, logit_v, sem0, sem1):
    nc = 2
    wid = lax.axis_index("s") * nc + lax.axis_index("c")

    pltpu.sync_copy(x_hbm.at[pl.ds(wid * RPW, RPW), :], x_v)
    pltpu.sync_copy(idx_hbm.at[wid], idx_v)

    def start_gather(t, wbuf, sem):
        pltpu.async_copy(w_hbm.at[idx_v.at[pl.ds(t * CH, CH)]], wbuf, sem)

    def wait_gather(t, wbuf, sem):
        pltpu.make_async_copy(
            w_hbm.at[idx_v.at[pl.ds(t * CH, CH)]], wbuf, sem).wait()

    start_gather(0, wbuf0, sem0)
    start_gather(1, wbuf1, sem1)

    lane = lax.iota(jnp.int32, 16)
    dnums = lax.GatherDimensionNumbers(
        offset_dims=(), collapsed_slice_dims=(0,), start_index_map=(0,))

    def lane_sum(v):
        for s in (8, 4, 2, 1):
            perm = lane ^ s
            v = v + lax.gather(
                v, perm[:, None], dnums, slice_sizes=(1,),
                mode=lax.GatherScatterMode.PROMISE_IN_BOUNDS)
        return v

    nsub = RPW // CH

    def group(g, _):
        for sub in range(nsub):
            t = g * nsub + sub
            wbuf = wbuf0 if sub % 2 == 0 else wbuf1
            sem = sem0 if sub % 2 == 0 else sem1
            xbase = sub * CH
            wait_gather(t, wbuf, sem)

            def blk(q, _, xbase=xbase, wbuf=wbuf):
                r0 = q * 16
                lvec = jnp.zeros((16,), jnp.float32)
                for l in range(16):
                    r = r0 + l
                    xr = xbase + r
                    p = []
                    for d in range(D // 32):
                        xb = plsc.bitcast(
                            x_v[xr, pl.ds(d * 16, 16)], jnp.bfloat16)
                        wb = plsc.bitcast(
                            wbuf[r, pl.ds(d * 16, 16)], jnp.bfloat16)
                        prod = xb * wb
                        pi = plsc.bitcast(prod, jnp.int32)
                        hi = plsc.bitcast(
                            pi & jnp.int32(-65536), jnp.float32)
                        lo = plsc.bitcast(pi << 16, jnp.float32)
                        p += [hi, lo]
                    while len(p) > 1:
                        p = [p[i] + p[i + 1] for i in range(0, len(p) - 1, 2)] \
                            + ([p[-1]] if len(p) % 2 else [])
                    lvec = jnp.where(lane == l, lane_sum(p[0]), lvec)
                logit_v[pl.ds(xbase + r0, 16)] = lvec
                return 0

            lax.fori_loop(0, CH // 16, blk, 0, unroll=2)

            if sub == nsub - 1:
                pltpu.sync_copy(logit_v, out_hbm.at[wid, g])
            if sub < 2:
                start_gather(t + 2, wbuf, sem)
            else:
                @pl.when(g < G - 1)
                def _():
                    start_gather(t + 2, wbuf, sem)
        return 0

    lax.fori_loop(0, G, group, 0)


_sc_logits = functools.partial(
    pl.kernel,
    mesh=plsc.VectorSubcoreMesh(core_axis_name="c", subcore_axis_name="s"),
    compiler_params=pltpu.CompilerParams(
        needs_layout_passes=False, use_tc_tiling_on_sc=False),
    out_type=jax.ShapeDtypeStruct((NW, G, RPW), jnp.float32),
    scratch_types=[
        pltpu.VMEM((RPW, D // 2), jnp.int32),
        pltpu.VMEM((G * RPW,), jnp.int32),
        pltpu.VMEM((CH, D // 2), jnp.int32),
        pltpu.VMEM((CH, D // 2), jnp.int32),
        pltpu.VMEM((RPW,), jnp.float32),
        pltpu.SemaphoreType.DMA,
        pltpu.SemaphoreType.DMA,
    ],
)(_sc_body)


def _tc_body(l_ref, out_ref):
    l = l_ref[...]
    sp = jnp.maximum(l, 0.0) + jnp.log1p(jnp.exp(-jnp.abs(l)))
    rows = lax.broadcasted_iota(jnp.int32, l.shape, 0)
    pos = (rows % (G * RPW // 128)) < (RPW // 128)
    total = jnp.sum(sp) - jnp.sum(jnp.where(pos, l, 0.0))
    out_ref[0, 0] = total / np.float32(T)


def _tc_loss(logits_flat):
    return pl.pallas_call(
        _tc_body,
        out_shape=jax.ShapeDtypeStruct((1, 1), jnp.float32),
        out_specs=pl.BlockSpec(memory_space=pltpu.SMEM),
    )(logits_flat)


def kernel(inputs, weights, labels, neg_num):
    neg = jax.random.randint(jax.random.key(1), (NEG * B,), 0, weights.shape[0])
    idx = jnp.concatenate([labels.astype(jnp.int32), neg.astype(jnp.int32)])
    idx_t = idx.reshape(G, NW, RPW).transpose(1, 0, 2).reshape(NW, G * RPW)
    xb = lax.bitcast_convert_type(
        inputs.astype(jnp.bfloat16).reshape(B, D // 2, 2), jnp.int32)
    wb = lax.bitcast_convert_type(
        weights.astype(jnp.bfloat16).reshape(weights.shape[0], D // 2, 2),
        jnp.int32)
    logits = _sc_logits(xb, wb, idx_t)
    loss = _tc_loss(logits.reshape(T // 128, 128))
    return loss[0, 0]


# speedup vs baseline: 2.6413x; 2.6413x over previous
import functools

import jax
import jax.numpy as jnp
import numpy as np
from jax import lax
from jax.experimental import pallas as pl
from jax.experimental.pallas import tpu as pltpu
from jax.experimental.pallas import tpu_sc as plsc

B = 16384
D = 128
NEG = 20
G = NEG + 1
T = G * B
NW = 32
RPW = B // NW
CH = 128
STEPS = G * (RPW // CH)
HD = D // 2


def _pack_body(w_ref, out_ref):
    w = w_ref[...]
    u = lax.bitcast_convert_type(w.astype(jnp.bfloat16), jnp.uint16)
    u = u.astype(jnp.int32)
    out_ref[...] = u[:, :HD] | (u[:, HD:] << 16)


def _pack_bf16_pairs(arr, rows_per_block):
    n = arr.shape[0]
    return pl.pallas_call(
        _pack_body,
        grid=(n // rows_per_block,),
        in_specs=[pl.BlockSpec((rows_per_block, D), lambda i: (i, 0))],
        out_specs=pl.BlockSpec((rows_per_block, HD), lambda i: (i, 0)),
        out_shape=jax.ShapeDtypeStruct((n, HD), jnp.int32),
    )(arr)


def _sc_body(x_hbm, w_hbm, lab_hbm, neg_hbm, out_hbm,
             x_v, idx_v, wbuf0, wbuf1, logit_v, sem0, sem1, isem):
    nc = 2
    wid = lax.axis_index("s") * nc + lax.axis_index("c")

    pltpu.sync_copy(x_hbm.at[pl.ds(wid * RPW, RPW), :], x_v)

    copies = [pltpu.make_async_copy(
        lab_hbm.at[pl.ds(wid * RPW, RPW)], idx_v.at[pl.ds(0, RPW)], isem)]
    for g in range(1, G):
        copies.append(pltpu.make_async_copy(
            neg_hbm.at[pl.ds((g - 1) * B + wid * RPW, RPW)],
            idx_v.at[pl.ds(g * RPW, RPW)], isem))
    for c in copies:
        c.start()
    for c in copies:
        c.wait()

    def start_gather(t, wbuf, sem):
        pltpu.async_copy(w_hbm.at[idx_v.at[pl.ds(t * CH, CH)]], wbuf, sem)

    def wait_gather(t, wbuf, sem):
        pltpu.make_async_copy(
            w_hbm.at[idx_v.at[pl.ds(t * CH, CH)]], wbuf, sem).wait()

    start_gather(0, wbuf0, sem0)
    start_gather(1, wbuf1, sem1)

    lane = lax.iota(jnp.int32, 16)
    dnums = lax.GatherDimensionNumbers(
        offset_dims=(), collapsed_slice_dims=(0,), start_index_map=(0,))

    def lane_sum(v):
        for s in (8, 4, 2, 1):
            perm = lane ^ s
            v = v + lax.gather(
                v, perm[:, None], dnums, slice_sizes=(1,),
                mode=lax.GatherScatterMode.PROMISE_IN_BOUNDS)
        return v

    nsub = RPW // CH

    def group(g, _):
        for sub in range(nsub):
            t = g * nsub + sub
            wbuf = wbuf0 if sub % 2 == 0 else wbuf1
            sem = sem0 if sub % 2 == 0 else sem1
            xbase = sub * CH
            wait_gather(t, wbuf, sem)

            def blk(q, _, xbase=xbase, wbuf=wbuf):
                r0 = q * 16
                lvec = jnp.zeros((16,), jnp.float32)
                for l in range(16):
                    r = r0 + l
                    xr = xbase + r
                    p = []
                    for d in range(HD // 16):
                        xb = plsc.bitcast(
                            x_v[xr, pl.ds(d * 16, 16)], jnp.bfloat16)
                        wb = plsc.bitcast(
                            wbuf[r, pl.ds(d * 16, 16)], jnp.bfloat16)
                        prod = xb * wb
                        pi = plsc.bitcast(prod, jnp.int32)
                        hi = plsc.bitcast(
                            pi & jnp.int32(-65536), jnp.float32)
                        lo = plsc.bitcast(pi << 16, jnp.float32)
                        p += [hi, lo]
                    while len(p) > 1:
                        p = [p[i] + p[i + 1] for i in range(0, len(p) - 1, 2)] \
                            + ([p[-1]] if len(p) % 2 else [])
                    lvec = jnp.where(lane == l, lane_sum(p[0]), lvec)
                logit_v[pl.ds(xbase + r0, 16)] = lvec
                return 0

            lax.fori_loop(0, CH // 16, blk, 0, unroll=2)

            if sub == nsub - 1:
                pltpu.sync_copy(logit_v, out_hbm.at[wid, g])
            if sub < 2:
                start_gather(t + 2, wbuf, sem)
            else:
                @pl.when(g < G - 1)
                def _():
                    start_gather(t + 2, wbuf, sem)
        return 0

    lax.fori_loop(0, G, group, 0)


_sc_logits = functools.partial(
    pl.kernel,
    mesh=plsc.VectorSubcoreMesh(core_axis_name="c", subcore_axis_name="s"),
    compiler_params=pltpu.CompilerParams(
        needs_layout_passes=False, use_tc_tiling_on_sc=False),
    out_type=jax.ShapeDtypeStruct((NW, G, RPW), jnp.float32),
    scratch_types=[
        pltpu.VMEM((RPW, HD), jnp.int32),
        pltpu.VMEM((G * RPW,), jnp.int32),
        pltpu.VMEM((CH, HD), jnp.int32),
        pltpu.VMEM((CH, HD), jnp.int32),
        pltpu.VMEM((RPW,), jnp.float32),
        pltpu.SemaphoreType.DMA,
        pltpu.SemaphoreType.DMA,
        pltpu.SemaphoreType.DMA,
    ],
)(_sc_body)


def _tc_body(l_ref, out_ref):
    l = l_ref[...]
    sp = jnp.maximum(l, 0.0) + jnp.log1p(jnp.exp(-jnp.abs(l)))
    rows = lax.broadcasted_iota(jnp.int32, l.shape, 0)
    pos = (rows % (G * RPW // 128)) < (RPW // 128)
    total = jnp.sum(sp) - jnp.sum(jnp.where(pos, l, 0.0))
    out_ref[0, 0] = total / np.float32(T)


def _tc_loss(logits_flat):
    return pl.pallas_call(
        _tc_body,
        out_shape=jax.ShapeDtypeStruct((1, 1), jnp.float32),
        out_specs=pl.BlockSpec(memory_space=pltpu.SMEM),
    )(logits_flat)


def kernel(inputs, weights, labels, neg_num):
    neg = jax.random.randint(
        jax.random.key(1), (NEG * B,), 0, weights.shape[0], dtype=jnp.int32)
    xb = _pack_bf16_pairs(inputs, 2048)
    wb = _pack_bf16_pairs(weights, 2000)
    logits = _sc_logits(xb, wb, labels.astype(jnp.int32), neg)
    loss = _tc_loss(logits.reshape(T // 128, 128))
    return loss[0, 0]
